# Initial kernel scaffold; baseline (speedup 1.0000x reference)
import jax
import jax.numpy as jnp
from jax.experimental import pallas as pl


def kernel(x, edge_index, W_enc, b_enc, W_dec, b_dec, W_conv, b_conv):
    raise NotImplementedError("write your pallas kernel here")


# speedup vs baseline: 6.7119x; 6.7119x over previous
import functools

import jax
import jax.numpy as jnp
from jax import lax
from jax.experimental import pallas as pl
from jax.experimental.pallas import tpu as pltpu
from jax.experimental.pallas import tpu_sc as plsc

N = 10000
E = 320000
D = 128
S = 64
STEPS = 8
DT = 0.125
EPS = 1e-12

NC = 2
NSUB = 16
NW = NC * NSUB
CH = 128
CPW = 79
EPAD = NW * CPW * CH
NOUT = 10240
RPT = NOUT // NSUB

RB = 1000
GRID = N // RB

A_ACC = (DT / 6.0, DT / 3.0, DT / 3.0, DT / 6.0)
A_Y = (DT / 2.0, DT / 2.0, DT, 0.0)


def _sc_agg(table, srcr, dstr):
    mesh = plsc.VectorSubcoreMesh(core_axis_name="c", subcore_axis_name="s")

    @functools.partial(
        pl.kernel,
        mesh=mesh,
        out_type=jax.ShapeDtypeStruct((NC, NOUT, S), jnp.float32),
        scratch_types=[
            pltpu.VMEM((CPW, CH), jnp.int32),
            pltpu.VMEM((CPW, CH), jnp.int32),
            pltpu.VMEM((CH, S), jnp.float32),
            pltpu.VMEM((RPT, S), jnp.float32),
            pltpu.VMEM_SHARED((NOUT, S), jnp.float32),
            pltpu.SemaphoreType.DMA,
        ],
        compiler_params=pltpu.CompilerParams(use_tc_tiling_on_sc=False),
    )
    def k(table_hbm, src_hbm, dst_hbm, out_hbm, src_v, dst_v, rows_v, stg_v,
          agg_sh, sem):
        c = lax.axis_index("c")
        s = lax.axis_index("s")
        wid = c * NSUB + s
        pltpu.sync_copy(src_hbm.at[wid], src_v)
        pltpu.sync_copy(dst_hbm.at[wid], dst_v)

        def zrow(i, carry):
            for g in range(S // 16):
                stg_v[i, pl.ds(g * 16, 16)] = jnp.zeros((16,), jnp.float32)
            return carry

        lax.fori_loop(0, RPT, zrow, 0)
        pltpu.sync_copy(stg_v, agg_sh.at[pl.ds(s * RPT, RPT)])
        plsc.subcore_barrier()

        def body(j, carry):
            pltpu.async_copy(table_hbm.at[src_v.at[j]], rows_v, sem).wait()
            pltpu.sync_copy(rows_v, agg_sh.at[dst_v.at[j]], add=True)
            return carry

        lax.fori_loop(0, CPW, body, 0)
        plsc.subcore_barrier()

        pltpu.sync_copy(agg_sh.at[pl.ds(s * RPT, RPT)], stg_v)
        pltpu.sync_copy(stg_v, out_hbm.at[c, pl.ds(s * RPT, RPT)])

    return k(table, srcr, dstr)


def _softmax(z):
    m = jnp.max(z, axis=-1, keepdims=True)
    ez = jnp.exp(z - m)
    return ez / jnp.sum(ez, axis=-1, keepdims=True)


def _enc_body(x_ref, we_ref, be_ref, y0_ref, p0_ref):
    enc = lax.dot_general(x_ref[...], we_ref[...], (((1,), (1,)), ((), ())),
                          preferred_element_type=jnp.float32) + be_ref[...]
    y0 = _softmax(enc)
    y0_ref[...] = y0
    p0_ref[...] = _softmax(y0)


def _tc_encode(x, W_enc, be1):
    return pl.pallas_call(
        _enc_body,
        grid=(GRID,),
        in_specs=[
            pl.BlockSpec((RB, D), lambda i: (i, 0)),
            pl.BlockSpec((S, D), lambda i: (0, 0)),
            pl.BlockSpec((1, S), lambda i: (0, 0)),
        ],
        out_specs=[pl.BlockSpec((RB, S), lambda i: (i, 0))] * 2,
        out_shape=[jax.ShapeDtypeStruct((N, S), jnp.float32)] * 2,
    )(x, W_enc, be1)


def _post_body(a_acc, a_y, last, yb_ref, ya_ref, p_ref, agg_ref, deg_ref,
               wc_ref, bc_ref, ya2_ref, p2_ref):
    agg = agg_ref[0] + agg_ref[1]
    deg = deg_ref[0, :, 0:1] + deg_ref[1, :, 0:1]
    aggn = agg / jnp.maximum(deg, 1.0)
    grad = lax.dot_general(aggn, wc_ref[...], (((1,), (1,)), ((), ())),
                           preferred_element_type=jnp.float32) + bc_ref[...]
    k = jnp.maximum(p_ref[...], EPS) * grad
    k = k - jnp.mean(k, axis=-1, keepdims=True)
    ya2 = ya_ref[...] + a_acc * k
    z = ya2 if last else yb_ref[...] + a_y * k
    ya2_ref[...] = ya2
    p2_ref[...] = _softmax(z)


def _tc_post(yb, ya, p, aggp, degp, W_conv, bc1, stage):
    body = functools.partial(_post_body, A_ACC[stage], A_Y[stage], stage == 3)
    return pl.pallas_call(
        body,
        grid=(GRID,),
        in_specs=[
            pl.BlockSpec((RB, S), lambda i: (i, 0)),
            pl.BlockSpec((RB, S), lambda i: (i, 0)),
            pl.BlockSpec((RB, S), lambda i: (i, 0)),
            pl.BlockSpec((NC, RB, S), lambda i: (0, i, 0)),
            pl.BlockSpec((NC, RB, S), lambda i: (0, i, 0)),
            pl.BlockSpec((S, S), lambda i: (0, 0)),
            pl.BlockSpec((1, S), lambda i: (0, 0)),
        ],
        out_specs=[pl.BlockSpec((RB, S), lambda i: (i, 0))] * 2,
        out_shape=[jax.ShapeDtypeStruct((N, S), jnp.float32)] * 2,
    )(yb, ya, p, aggp, degp, W_conv, bc1)


def _dec_body(y_ref, wd_ref, bd_ref, out_ref):
    out_ref[...] = lax.dot_general(
        y_ref[...], wd_ref[...], (((1,), (1,)), ((), ())),
        preferred_element_type=jnp.float32) + bd_ref[...]


def _tc_decode(y, W_dec, bd1):
    return pl.pallas_call(
        _dec_body,
        grid=(GRID,),
        in_specs=[
            pl.BlockSpec((RB, S), lambda i: (i, 0)),
            pl.BlockSpec((D, S), lambda i: (0, 0)),
            pl.BlockSpec((1, D), lambda i: (0, 0)),
        ],
        out_specs=pl.BlockSpec((RB, D), lambda i: (i, 0)),
        out_shape=jax.ShapeDtypeStruct((N, D), jnp.float32),
    )(y, W_dec, bd1)


def kernel(x, edge_index, W_enc, b_enc, W_dec, b_dec, W_conv, b_conv):
    src = edge_index[0]
    dst = edge_index[1]
    pad = EPAD - E
    srcr = jnp.concatenate([src, jnp.zeros((pad,), jnp.int32)]).reshape(
        NW, CPW, CH)
    dstr = jnp.concatenate([dst, jnp.full((pad,), N, jnp.int32)]).reshape(
        NW, CPW, CH)
    be1 = b_enc.reshape(1, S)
    bc1 = b_conv.reshape(1, S)
    bd1 = b_dec.reshape(1, D)

    degp = _sc_agg(jnp.ones((N, S), jnp.float32), srcr, dstr)
    y0, probs = _tc_encode(x, W_enc, be1)
    yb = y0
    ya = y0
    for _step in range(STEPS):
        for stage in range(4):
            aggp = _sc_agg(probs, srcr, dstr)
            ya, probs = _tc_post(yb, ya, probs, aggp, degp, W_conv, bc1,
                                 stage)
        yb = ya
    return _tc_decode(ya, W_dec, bd1)
